# baseline (speedup 1.0000x reference)
import functools

import jax
import jax.numpy as jnp
from jax import lax
from jax.experimental import pallas as pl
from jax.experimental.pallas import tpu as pltpu
from jax.experimental.pallas import tpu_sc as plsc

N = 10000
E = 320000
F_IN = 128
HID = 256
C = 40
CP = 48

NC = 2
NS = 16
NW = NC * NS
K = 112
NCHUNK = 90
EPT = NCHUNK * K
EPAD = NW * EPT - E
NP = N + 16

_MESH = dict(core_axis_name="c", subcore_axis_name="s",
             num_cores=NC, num_subcores=NS)


def _wid():
    return lax.axis_index("s") * NC + lax.axis_index("c")


_DEGQ = 8


def _deg_body(dst_hbm, ones_hbm, zeros_hbm, out_hbm, idx_v, ones_v, acc, sem):
    c = lax.axis_index("c")
    s = lax.axis_index("s")
    wid = _wid()

    @pl.when(s == 0)
    def _():
        pltpu.sync_copy(zeros_hbm, acc)

    pltpu.sync_copy(ones_hbm, ones_v)
    pltpu.sync_copy(dst_hbm.at[wid], idx_v)
    plsc.subcore_barrier()

    def body(j, carry):
        @pl.when(j >= _DEGQ)
        def _():
            pltpu.make_async_copy(ones_v, acc.at[idx_v.at[0]], sem).wait()

        pltpu.async_copy(ones_v, acc.at[idx_v.at[j]], sem, add=True)
        return carry

    lax.fori_loop(0, NCHUNK, body, 0)
    for _ in range(_DEGQ):
        pltpu.make_async_copy(ones_v, acc.at[idx_v.at[0]], sem).wait()
    plsc.subcore_barrier()

    @pl.when(s == 0)
    def _():
        pltpu.sync_copy(acc.at[pl.ds(0, N)], out_hbm.at[c])


def _make_deg_kernel():
    return pl.kernel(
        _deg_body,
        out_type=jax.ShapeDtypeStruct((NC, N, 16), jnp.float32),
        mesh=plsc.VectorSubcoreMesh(**_MESH),
        compiler_params=pltpu.CompilerParams(use_tc_tiling_on_sc=False),
        scratch_types=[
            pltpu.VMEM((NCHUNK, K), jnp.int32),
            pltpu.VMEM((K, 16), jnp.float32),
            pltpu.VMEM_SHARED((NP, 16), jnp.float32),
            pltpu.SemaphoreType.DMA,
        ],
    )


def _agg_body(nbuf, u_hbm, src_hbm, dst_hbm, zeros_hbm, out_hbm,
              idx_s, idx_d, rows_v, acc, gsem, ssem):
    c = lax.axis_index("c")
    s = lax.axis_index("s")
    wid = _wid()

    @pl.when(s == 0)
    def _():
        pltpu.sync_copy(zeros_hbm, acc)

    pltpu.sync_copy(src_hbm.at[wid], idx_s)
    pltpu.sync_copy(dst_hbm.at[wid], idx_d)
    plsc.subcore_barrier()

    def body(j, carry):
        b = lax.rem(j, nbuf)

        @pl.when(j < NCHUNK)
        def _():
            @pl.when(j >= nbuf)
            def _():
                pltpu.make_async_copy(
                    rows_v.at[b], acc.at[idx_d.at[0]], ssem).wait()

            pltpu.async_copy(u_hbm.at[idx_s.at[j]], rows_v.at[b], gsem)

        @pl.when(j >= 1)
        def _():
            pb = lax.rem(j - 1, nbuf)
            pltpu.make_async_copy(
                u_hbm.at[idx_s.at[0]], rows_v.at[pb], gsem).wait()
            pltpu.async_copy(
                rows_v.at[pb], acc.at[idx_d.at[j - 1]], ssem, add=True)

        return carry

    lax.fori_loop(0, NCHUNK + 1, body, 0)
    for _ in range(nbuf):
        pltpu.make_async_copy(rows_v.at[0], acc.at[idx_d.at[0]], ssem).wait()
    plsc.subcore_barrier()

    @pl.when(s == 0)
    def _():
        pltpu.sync_copy(acc.at[pl.ds(0, N)], out_hbm.at[c])


def _make_agg_kernel(d, nbuf):
    return pl.kernel(
        functools.partial(_agg_body, nbuf),
        out_type=jax.ShapeDtypeStruct((NC, N, d), jnp.float32),
        mesh=plsc.VectorSubcoreMesh(**_MESH),
        compiler_params=pltpu.CompilerParams(use_tc_tiling_on_sc=False),
        scratch_types=[
            pltpu.VMEM((NCHUNK, K), jnp.int32),
            pltpu.VMEM((NCHUNK, K), jnp.int32),
            pltpu.VMEM((nbuf, K, d), jnp.float32),
            pltpu.VMEM_SHARED((NP, d), jnp.float32),
            pltpu.SemaphoreType.DMA,
            pltpu.SemaphoreType.DMA,
        ],
    )


def _scale_body(degp_ref, x_ref, dinv_ref, u1_ref):
    deg = degp_ref[0, :, 0:1] + degp_ref[1, :, 0:1] + 1.0
    dinv = lax.rsqrt(deg)
    dinv_ref[...] = jnp.broadcast_to(dinv, (N, 16))
    u1_ref[...] = x_ref[...] * dinv


def _tc_scale(degp, x):
    return pl.pallas_call(
        _scale_body,
        out_shape=(
            jax.ShapeDtypeStruct((N, 16), jnp.float32),
            jax.ShapeDtypeStruct((N, F_IN), jnp.float32),
        ),
    )(degp, x)


def _mlp_body(s1p_ref, u1_ref, dinv_ref, w1_ref, b1_ref, w2_ref, u2_ref):
    dv = dinv_ref[:, 0:1]
    agg1 = dv * (s1p_ref[0] + s1p_ref[1] + u1_ref[...])
    h = jnp.maximum(
        jnp.dot(agg1, w1_ref[...], preferred_element_type=jnp.float32,
                precision=lax.Precision.HIGHEST) + b1_ref[...],
        0.0)
    z = jnp.dot(h, w2_ref[...], preferred_element_type=jnp.float32,
                precision=lax.Precision.HIGHEST)
    u2_ref[...] = dv * z


def _tc_mlp(s1p, u1, dinv, W1, b1_2d, W2p):
    return pl.pallas_call(
        _mlp_body,
        out_shape=jax.ShapeDtypeStruct((N, CP), jnp.float32),
    )(s1p, u1, dinv, W1, b1_2d, W2p)


def _loss_body(s2p_ref, u2_ref, dinv_ref, b2_ref, y_ref, maskf_ref, w_ref,
               out_ref):
    dv = dinv_ref[:, 0:1]
    logits = dv * (s2p_ref[0] + s2p_ref[1] + u2_ref[...]) + b2_ref[...]
    col = lax.broadcasted_iota(jnp.int32, (N, CP), 1)
    valid = col < C
    neg = jnp.where(valid, logits, -1e30)
    m = jnp.max(neg, axis=1, keepdims=True)
    ex = jnp.where(valid, jnp.exp(logits - m), 0.0)
    lse = jnp.log(jnp.sum(ex, axis=1, keepdims=True)) + m
    onehot = col == y_ref[...]
    picked = jnp.sum(jnp.where(onehot, logits, 0.0), axis=1, keepdims=True)
    wy = jnp.sum(jnp.where(onehot, w_ref[...], 0.0), axis=1, keepdims=True)
    nll = lse - picked
    wv = wy * maskf_ref[...]
    num = jnp.sum(nll * wv, axis=0, keepdims=True)
    den = jnp.sum(wv, axis=0, keepdims=True)
    out_ref[...] = num / den


def _tc_loss(s2p, u2, dinv, b2p, y2, maskf, wpad):
    return pl.pallas_call(
        _loss_body,
        out_shape=jax.ShapeDtypeStruct((1, 1), jnp.float32),
    )(s2p, u2, dinv, b2p, y2, maskf, wpad)


def kernel(x, edge_index, y, mask, weight, W1, b1, W2, b2):
    src = jnp.concatenate(
        [edge_index[0].astype(jnp.int32), jnp.zeros((EPAD,), jnp.int32)]
    ).reshape(NW, NCHUNK, K)
    dst = jnp.concatenate(
        [edge_index[1].astype(jnp.int32), jnp.full((EPAD,), N, jnp.int32)]
    ).reshape(NW, NCHUNK, K)

    zeros16 = jnp.zeros((NP, 16), jnp.float32)
    zerosF = jnp.zeros((NP, F_IN), jnp.float32)
    zerosC = jnp.zeros((NP, CP), jnp.float32)
    onesK = jnp.ones((K, 16), jnp.float32)

    degp = _make_deg_kernel()(dst, onesK, zeros16)
    dinv, u1 = _tc_scale(degp, x)

    s1p = _make_agg_kernel(F_IN, 2)(u1, src, dst, zerosF)

    W2p = jnp.pad(W2, ((0, 0), (0, CP - C)))
    u2 = _tc_mlp(s1p, u1, dinv, W1, b1.reshape(1, HID), W2p)

    s2p = _make_agg_kernel(CP, 8)(u2, src, dst, zerosC)

    b2p = jnp.pad(b2, (0, CP - C)).reshape(1, CP)
    wpad = jnp.pad(weight, (0, CP - C)).reshape(1, CP)
    y2 = y.astype(jnp.int32).reshape(N, 1)
    maskf = mask.astype(jnp.float32).reshape(N, 1)

    loss = _tc_loss(s2p, u2, dinv, b2p, y2, maskf, wpad)
    return jnp.reshape(loss, ())


# speedup vs baseline: 1.4522x; 1.4522x over previous
import functools

import jax
import jax.numpy as jnp
from jax import lax
from jax.experimental import pallas as pl
from jax.experimental.pallas import tpu as pltpu
from jax.experimental.pallas import tpu_sc as plsc

N = 10000
E = 320000
F_IN = 128
HID = 256
C = 40
CP = 48

NC = 2
NS = 16
NW = NC * NS
K = 80
NCHUNK = 125
EPT = NCHUNK * K
EPAD = NW * EPT - E
NP = N + 16

_MESH = dict(core_axis_name="c", subcore_axis_name="s",
             num_cores=NC, num_subcores=NS)


def _wid():
    return lax.axis_index("s") * NC + lax.axis_index("c")


_DEGQ = 8


def _deg_body(dst_hbm, ones_hbm, zeros_hbm, out_hbm, idx_v, ones_v, acc, sem):
    c = lax.axis_index("c")
    s = lax.axis_index("s")
    wid = _wid()

    @pl.when(s == 0)
    def _():
        pltpu.sync_copy(zeros_hbm, acc)

    pltpu.sync_copy(ones_hbm, ones_v)
    pltpu.sync_copy(dst_hbm.at[wid], idx_v)
    plsc.subcore_barrier()

    def body(j, carry):
        @pl.when(j >= _DEGQ)
        def _():
            pltpu.make_async_copy(ones_v, acc.at[idx_v.at[0]], sem).wait()

        pltpu.async_copy(ones_v, acc.at[idx_v.at[j]], sem, add=True)
        return carry

    lax.fori_loop(0, NCHUNK, body, 0)
    for _ in range(_DEGQ):
        pltpu.make_async_copy(ones_v, acc.at[idx_v.at[0]], sem).wait()
    plsc.subcore_barrier()

    @pl.when(s == 0)
    def _():
        pltpu.sync_copy(acc.at[pl.ds(0, N)], out_hbm.at[c])


def _make_deg_kernel():
    return pl.kernel(
        _deg_body,
        out_type=jax.ShapeDtypeStruct((NC, N, 16), jnp.float32),
        mesh=plsc.VectorSubcoreMesh(**_MESH),
        compiler_params=pltpu.CompilerParams(use_tc_tiling_on_sc=False),
        scratch_types=[
            pltpu.VMEM((NCHUNK, K), jnp.int32),
            pltpu.VMEM((K, 16), jnp.float32),
            pltpu.VMEM_SHARED((NP, 16), jnp.float32),
            pltpu.SemaphoreType.DMA,
        ],
    )


def _agg_body(nbuf, u_hbm, src_hbm, dst_hbm, zeros_hbm, out_hbm,
              idx_s, idx_d, rows_v, acc, gsem, ssem):
    c = lax.axis_index("c")
    s = lax.axis_index("s")
    wid = _wid()

    @pl.when(s == 0)
    def _():
        pltpu.sync_copy(zeros_hbm, acc)

    pltpu.sync_copy(src_hbm.at[wid], idx_s)
    pltpu.sync_copy(dst_hbm.at[wid], idx_d)
    plsc.subcore_barrier()

    def body(j, carry):
        b = lax.rem(j, nbuf)

        @pl.when(j < NCHUNK)
        def _():
            @pl.when(j >= nbuf)
            def _():
                pltpu.make_async_copy(
                    rows_v.at[b], acc.at[idx_d.at[0]], ssem).wait()

            pltpu.async_copy(u_hbm.at[idx_s.at[j]], rows_v.at[b], gsem)

        @pl.when(j >= 1)
        def _():
            pb = lax.rem(j - 1, nbuf)
            pltpu.make_async_copy(
                u_hbm.at[idx_s.at[0]], rows_v.at[pb], gsem).wait()
            pltpu.async_copy(
                rows_v.at[pb], acc.at[idx_d.at[j - 1]], ssem, add=True)

        return carry

    lax.fori_loop(0, NCHUNK + 1, body, 0)
    for _ in range(nbuf):
        pltpu.make_async_copy(rows_v.at[0], acc.at[idx_d.at[0]], ssem).wait()
    plsc.subcore_barrier()

    @pl.when(s == 0)
    def _():
        pltpu.sync_copy(acc.at[pl.ds(0, N)], out_hbm.at[c])


def _make_agg_kernel(d, nbuf):
    return pl.kernel(
        functools.partial(_agg_body, nbuf),
        out_type=jax.ShapeDtypeStruct((NC, N, d), jnp.float32),
        mesh=plsc.VectorSubcoreMesh(**_MESH),
        compiler_params=pltpu.CompilerParams(use_tc_tiling_on_sc=False),
        scratch_types=[
            pltpu.VMEM((NCHUNK, K), jnp.int32),
            pltpu.VMEM((NCHUNK, K), jnp.int32),
            pltpu.VMEM((nbuf, K, d), jnp.float32),
            pltpu.VMEM_SHARED((NP, d), jnp.float32),
            pltpu.SemaphoreType.DMA,
            pltpu.SemaphoreType.DMA,
        ],
    )


def _scale_body(degp_ref, x_ref, dinv_ref, u1_ref):
    deg = degp_ref[0, :, 0:1] + degp_ref[1, :, 0:1] + 1.0
    dinv = lax.rsqrt(deg)
    dinv_ref[...] = jnp.broadcast_to(dinv, (N, 16))
    u1_ref[...] = x_ref[...] * dinv


def _tc_scale(degp, x):
    return pl.pallas_call(
        _scale_body,
        out_shape=(
            jax.ShapeDtypeStruct((N, 16), jnp.float32),
            jax.ShapeDtypeStruct((N, F_IN), jnp.float32),
        ),
    )(degp, x)


def _mlp_body(s1p_ref, u1_ref, dinv_ref, w1_ref, b1_ref, w2_ref, u2_ref):
    dv = dinv_ref[:, 0:1]
    agg1 = dv * (s1p_ref[0] + s1p_ref[1] + u1_ref[...])
    h = jnp.maximum(
        jnp.dot(agg1, w1_ref[...], preferred_element_type=jnp.float32,
                precision=lax.Precision.HIGHEST) + b1_ref[...],
        0.0)
    z = jnp.dot(h, w2_ref[...], preferred_element_type=jnp.float32,
                precision=lax.Precision.HIGHEST)
    u2_ref[...] = dv * z


def _tc_mlp(s1p, u1, dinv, W1, b1_2d, W2p):
    return pl.pallas_call(
        _mlp_body,
        out_shape=jax.ShapeDtypeStruct((N, CP), jnp.float32),
    )(s1p, u1, dinv, W1, b1_2d, W2p)


def _loss_body(s2p_ref, u2_ref, dinv_ref, b2_ref, y_ref, maskf_ref, w_ref,
               out_ref):
    dv = dinv_ref[:, 0:1]
    logits = dv * (s2p_ref[0] + s2p_ref[1] + u2_ref[...]) + b2_ref[...]
    col = lax.broadcasted_iota(jnp.int32, (N, CP), 1)
    valid = col < C
    neg = jnp.where(valid, logits, -1e30)
    m = jnp.max(neg, axis=1, keepdims=True)
    ex = jnp.where(valid, jnp.exp(logits - m), 0.0)
    lse = jnp.log(jnp.sum(ex, axis=1, keepdims=True)) + m
    onehot = col == y_ref[...]
    picked = jnp.sum(jnp.where(onehot, logits, 0.0), axis=1, keepdims=True)
    wy = jnp.sum(jnp.where(onehot, w_ref[...], 0.0), axis=1, keepdims=True)
    nll = lse - picked
    wv = wy * maskf_ref[...]
    num = jnp.sum(nll * wv, axis=0, keepdims=True)
    den = jnp.sum(wv, axis=0, keepdims=True)
    out_ref[...] = num / den


def _tc_loss(s2p, u2, dinv, b2p, y2, maskf, wpad):
    return pl.pallas_call(
        _loss_body,
        out_shape=jax.ShapeDtypeStruct((1, 1), jnp.float32),
    )(s2p, u2, dinv, b2p, y2, maskf, wpad)


def kernel(x, edge_index, y, mask, weight, W1, b1, W2, b2):
    src = jnp.concatenate(
        [edge_index[0].astype(jnp.int32), jnp.zeros((EPAD,), jnp.int32)]
    ).reshape(NW, NCHUNK, K)
    dst = jnp.concatenate(
        [edge_index[1].astype(jnp.int32), jnp.full((EPAD,), N, jnp.int32)]
    ).reshape(NW, NCHUNK, K)

    zeros16 = jnp.zeros((NP, 16), jnp.float32)
    zerosF = jnp.zeros((NP, F_IN), jnp.float32)
    zerosC = jnp.zeros((NP, CP), jnp.float32)
    onesK = jnp.ones((K, 16), jnp.float32)

    degp = _make_deg_kernel()(dst, onesK, zeros16)
    dinv, u1 = _tc_scale(degp, x)

    s1p = _make_agg_kernel(F_IN, 3)(u1, src, dst, zerosF)

    W2p = jnp.pad(W2, ((0, 0), (0, CP - C)))
    u2 = _tc_mlp(s1p, u1, dinv, W1, b1.reshape(1, HID), W2p)

    s2p = _make_agg_kernel(CP, 12)(u2, src, dst, zerosC)

    b2p = jnp.pad(b2, (0, CP - C)).reshape(1, CP)
    wpad = jnp.pad(weight, (0, CP - C)).reshape(1, CP)
    y2 = y.astype(jnp.int32).reshape(N, 1)
    maskf = mask.astype(jnp.float32).reshape(N, 1)

    loss = _tc_loss(s2p, u2, dinv, b2p, y2, maskf, wpad)
    return jnp.reshape(loss, ())


# speedup vs baseline: 1.6469x; 1.1341x over previous
import functools

import jax
import jax.numpy as jnp
from jax import lax
from jax.experimental import pallas as pl
from jax.experimental.pallas import tpu as pltpu
from jax.experimental.pallas import tpu_sc as plsc

N = 10000
E = 320000
F_IN = 128
HID = 256
C = 40
CP = 48

NC = 2
NS = 16
NW = NC * NS
K = 80
NCHUNK = 125
EPT = NCHUNK * K
NP = N + 16

_MESH = dict(core_axis_name="c", subcore_axis_name="s",
             num_cores=NC, num_subcores=NS)


def _wid():
    return lax.axis_index("s") * NC + lax.axis_index("c")


_DEGQ = 8


def _deg_body(ei_hbm, ones_hbm, zeros_hbm, out_hbm, idx_v, ones_v, acc, sem):
    c = lax.axis_index("c")
    s = lax.axis_index("s")
    wid = _wid()

    @pl.when(s == 0)
    def _():
        pltpu.sync_copy(zeros_hbm, acc)

    pltpu.sync_copy(ones_hbm, ones_v)
    pltpu.sync_copy(ei_hbm.at[1, wid], idx_v)
    plsc.subcore_barrier()

    def body(j, carry):
        @pl.when(j >= _DEGQ)
        def _():
            pltpu.make_async_copy(ones_v, acc.at[idx_v.at[0]], sem).wait()

        pltpu.async_copy(ones_v, acc.at[idx_v.at[j]], sem, add=True)
        return carry

    lax.fori_loop(0, NCHUNK, body, 0)
    for _ in range(_DEGQ):
        pltpu.make_async_copy(ones_v, acc.at[idx_v.at[0]], sem).wait()
    plsc.subcore_barrier()

    @pl.when(s == 0)
    def _():
        pltpu.sync_copy(acc.at[pl.ds(0, N)], out_hbm.at[c])


def _make_deg_kernel():
    return pl.kernel(
        _deg_body,
        out_type=jax.ShapeDtypeStruct((NC, N, 16), jnp.float32),
        mesh=plsc.VectorSubcoreMesh(**_MESH),
        compiler_params=pltpu.CompilerParams(use_tc_tiling_on_sc=False),
        scratch_types=[
            pltpu.VMEM((NCHUNK, K), jnp.int32),
            pltpu.VMEM((K, 16), jnp.float32),
            pltpu.VMEM_SHARED((NP, 16), jnp.float32),
            pltpu.SemaphoreType.DMA,
        ],
    )


def _agg_body(nbuf, u_hbm, ei_hbm, zeros_hbm, out_hbm,
              idx_s, idx_d, rows_v, acc, gsem, ssem):
    c = lax.axis_index("c")
    s = lax.axis_index("s")
    wid = _wid()

    @pl.when(s == 0)
    def _():
        pltpu.sync_copy(zeros_hbm, acc)

    pltpu.sync_copy(ei_hbm.at[0, wid], idx_s)
    pltpu.sync_copy(ei_hbm.at[1, wid], idx_d)
    plsc.subcore_barrier()

    def body(j, carry):
        b = lax.rem(j, nbuf)

        @pl.when(j < NCHUNK)
        def _():
            @pl.when(j >= nbuf)
            def _():
                pltpu.make_async_copy(
                    rows_v.at[b], acc.at[idx_d.at[0]], ssem).wait()

            pltpu.async_copy(u_hbm.at[idx_s.at[j]], rows_v.at[b], gsem)

        @pl.when(j >= 1)
        def _():
            pb = lax.rem(j - 1, nbuf)
            pltpu.make_async_copy(
                u_hbm.at[idx_s.at[0]], rows_v.at[pb], gsem).wait()
            pltpu.async_copy(
                rows_v.at[pb], acc.at[idx_d.at[j - 1]], ssem, add=True)

        return carry

    lax.fori_loop(0, NCHUNK + 1, body, 0)
    for _ in range(nbuf):
        pltpu.make_async_copy(rows_v.at[0], acc.at[idx_d.at[0]], ssem).wait()
    plsc.subcore_barrier()

    @pl.when(s == 0)
    def _():
        pltpu.sync_copy(acc.at[pl.ds(0, N)], out_hbm.at[c])


def _make_agg_kernel(d, nbuf):
    return pl.kernel(
        functools.partial(_agg_body, nbuf),
        out_type=jax.ShapeDtypeStruct((NC, N, d), jnp.float32),
        mesh=plsc.VectorSubcoreMesh(**_MESH),
        compiler_params=pltpu.CompilerParams(use_tc_tiling_on_sc=False),
        scratch_types=[
            pltpu.VMEM((NCHUNK, K), jnp.int32),
            pltpu.VMEM((NCHUNK, K), jnp.int32),
            pltpu.VMEM((nbuf, K, d), jnp.float32),
            pltpu.VMEM_SHARED((NP, d), jnp.float32),
            pltpu.SemaphoreType.DMA,
            pltpu.SemaphoreType.DMA,
        ],
    )


def _scale_body(degp_ref, x_ref, dinv_ref, u1_ref):
    deg = degp_ref[0, :, 0:1] + degp_ref[1, :, 0:1] + 1.0
    dinv = lax.rsqrt(deg)
    dinv_ref[...] = jnp.broadcast_to(dinv, (N, 16))
    u1_ref[...] = x_ref[...] * dinv


def _tc_scale(degp, x):
    return pl.pallas_call(
        _scale_body,
        out_shape=(
            jax.ShapeDtypeStruct((N, 16), jnp.float32),
            jax.ShapeDtypeStruct((N, F_IN), jnp.float32),
        ),
    )(degp, x)


def _mlp_body(s1p_ref, u1_ref, dinv_ref, w1_ref, b1_ref, w2_ref, u2_ref):
    dv = dinv_ref[:, 0:1]
    agg1 = dv * (s1p_ref[0] + s1p_ref[1] + u1_ref[...])
    h = jnp.maximum(
        jnp.dot(agg1, w1_ref[...], preferred_element_type=jnp.float32,
                ) + b1_ref[...],
        0.0)
    z = jnp.dot(h, w2_ref[...], preferred_element_type=jnp.float32)
    u2_ref[...] = dv * z


def _tc_mlp(s1p, u1, dinv, W1, b1_2d, W2p):
    return pl.pallas_call(
        _mlp_body,
        out_shape=jax.ShapeDtypeStruct((N, CP), jnp.float32),
    )(s1p, u1, dinv, W1, b1_2d, W2p)


def _loss_body(s2p_ref, u2_ref, dinv_ref, b2_ref, y_ref, maskf_ref, w_ref,
               out_ref):
    dv = dinv_ref[:, 0:1]
    logits = dv * (s2p_ref[0] + s2p_ref[1] + u2_ref[...]) + b2_ref[...]
    col = lax.broadcasted_iota(jnp.int32, (N, CP), 1)
    valid = col < C
    neg = jnp.where(valid, logits, -1e30)
    m = jnp.max(neg, axis=1, keepdims=True)
    ex = jnp.where(valid, jnp.exp(logits - m), 0.0)
    lse = jnp.log(jnp.sum(ex, axis=1, keepdims=True)) + m
    onehot = col == y_ref[...]
    picked = jnp.sum(jnp.where(onehot, logits, 0.0), axis=1, keepdims=True)
    wy = jnp.sum(jnp.where(onehot, w_ref[...], 0.0), axis=1, keepdims=True)
    nll = lse - picked
    wv = wy * maskf_ref[...]
    num = jnp.sum(nll * wv, axis=0, keepdims=True)
    den = jnp.sum(wv, axis=0, keepdims=True)
    out_ref[...] = num / den


def _tc_loss(s2p, u2, dinv, b2p, y2, maskf, wpad):
    return pl.pallas_call(
        _loss_body,
        out_shape=jax.ShapeDtypeStruct((1, 1), jnp.float32),
    )(s2p, u2, dinv, b2p, y2, maskf, wpad)


def kernel(x, edge_index, y, mask, weight, W1, b1, W2, b2):
    ei = edge_index.astype(jnp.int32).reshape(2, NW, NCHUNK, K)

    zeros16 = jnp.zeros((NP, 16), jnp.float32)
    zerosF = jnp.zeros((NP, F_IN), jnp.float32)
    zerosC = jnp.zeros((NP, CP), jnp.float32)
    onesK = jnp.ones((K, 16), jnp.float32)

    degp = _make_deg_kernel()(ei, onesK, zeros16)
    dinv, u1 = _tc_scale(degp, x)

    s1p = _make_agg_kernel(F_IN, 3)(u1, ei, zerosF)

    W2p = jnp.pad(W2, ((0, 0), (0, CP - C)))
    u2 = _tc_mlp(s1p, u1, dinv, W1, b1.reshape(1, HID), W2p)

    s2p = _make_agg_kernel(CP, 12)(u2, ei, zerosC)

    b2p = jnp.pad(b2, (0, CP - C)).reshape(1, CP)
    wpad = jnp.pad(weight, (0, CP - C)).reshape(1, CP)
    y2 = y.astype(jnp.int32).reshape(N, 1)
    maskf = mask.astype(jnp.float32).reshape(N, 1)

    loss = _tc_loss(s2p, u2, dinv, b2p, y2, maskf, wpad)
    return jnp.reshape(loss, ())


# speedup vs baseline: 1.9658x; 1.1936x over previous
import functools

import jax
import jax.numpy as jnp
from jax import lax
from jax.experimental import pallas as pl
from jax.experimental.pallas import tpu as pltpu
from jax.experimental.pallas import tpu_sc as plsc

N = 10000
E = 320000
F_IN = 128
HID = 256
C = 40
CP = 48

NC = 2
NS = 16
NW = NC * NS
K = 80
NCHUNK = 125
EPT = NCHUNK * K
NP = N + 16

_MESH = dict(core_axis_name="c", subcore_axis_name="s",
             num_cores=NC, num_subcores=NS)


def _wid():
    return lax.axis_index("s") * NC + lax.axis_index("c")


_DEGQ = 8


def _deg_body(ei_hbm, ones_hbm, zeros_hbm, out_hbm, idx_v, ones_v, acc, sem):
    c = lax.axis_index("c")
    s = lax.axis_index("s")
    wid = _wid()

    @pl.when(s == 0)
    def _():
        pltpu.sync_copy(zeros_hbm, acc)

    pltpu.sync_copy(ones_hbm, ones_v)
    pltpu.sync_copy(ei_hbm.at[1, wid], idx_v)
    plsc.subcore_barrier()

    def body(j, carry):
        @pl.when(j >= _DEGQ)
        def _():
            pltpu.make_async_copy(ones_v, acc.at[idx_v.at[0]], sem).wait()

        pltpu.async_copy(ones_v, acc.at[idx_v.at[j]], sem, add=True)
        return carry

    lax.fori_loop(0, NCHUNK, body, 0)
    for _ in range(_DEGQ):
        pltpu.make_async_copy(ones_v, acc.at[idx_v.at[0]], sem).wait()
    plsc.subcore_barrier()

    @pl.when(s == 0)
    def _():
        pltpu.sync_copy(acc.at[pl.ds(0, N)], out_hbm.at[c])


def _make_deg_kernel():
    return pl.kernel(
        _deg_body,
        out_type=jax.ShapeDtypeStruct((NC, N, 16), jnp.float32),
        mesh=plsc.VectorSubcoreMesh(**_MESH),
        compiler_params=pltpu.CompilerParams(use_tc_tiling_on_sc=False),
        scratch_types=[
            pltpu.VMEM((NCHUNK, K), jnp.int32),
            pltpu.VMEM((K, 16), jnp.float32),
            pltpu.VMEM_SHARED((NP, 16), jnp.float32),
            pltpu.SemaphoreType.DMA,
        ],
    )


def _agg_body(nbuf, u_hbm, ei_hbm, zeros_hbm, out_hbm,
              idx_s, idx_d, rows_v, acc, gsem, ssem):
    c = lax.axis_index("c")
    s = lax.axis_index("s")
    wid = _wid()

    @pl.when(s == 0)
    def _():
        pltpu.sync_copy(zeros_hbm, acc)

    pltpu.sync_copy(ei_hbm.at[0, wid], idx_s)
    pltpu.sync_copy(ei_hbm.at[1, wid], idx_d)
    plsc.subcore_barrier()

    glag = max(1, nbuf // 2)

    def body(j, carry):
        b = lax.rem(j, nbuf)

        @pl.when(j < NCHUNK)
        def _():
            @pl.when(j >= nbuf)
            def _():
                pltpu.make_async_copy(
                    rows_v.at[b], acc.at[idx_d.at[0]], ssem).wait()

            pltpu.async_copy(u_hbm.at[idx_s.at[j]], rows_v.at[b], gsem)

        @pl.when(j >= glag)
        def _():
            pb = lax.rem(j - glag, nbuf)
            pltpu.make_async_copy(
                u_hbm.at[idx_s.at[0]], rows_v.at[pb], gsem).wait()
            pltpu.async_copy(
                rows_v.at[pb], acc.at[idx_d.at[j - glag]], ssem, add=True)

        return carry

    lax.fori_loop(0, NCHUNK + glag, body, 0)
    for _ in range(nbuf):
        pltpu.make_async_copy(rows_v.at[0], acc.at[idx_d.at[0]], ssem).wait()
    plsc.subcore_barrier()

    @pl.when(s == 0)
    def _():
        pltpu.sync_copy(acc.at[pl.ds(0, N)], out_hbm.at[c])


def _make_agg_kernel(d, nbuf):
    return pl.kernel(
        functools.partial(_agg_body, nbuf),
        out_type=jax.ShapeDtypeStruct((NC, N, d), jnp.float32),
        mesh=plsc.VectorSubcoreMesh(**_MESH),
        compiler_params=pltpu.CompilerParams(use_tc_tiling_on_sc=False),
        scratch_types=[
            pltpu.VMEM((NCHUNK, K), jnp.int32),
            pltpu.VMEM((NCHUNK, K), jnp.int32),
            pltpu.VMEM((nbuf, K, d), jnp.float32),
            pltpu.VMEM_SHARED((NP, d), jnp.float32),
            pltpu.SemaphoreType.DMA,
            pltpu.SemaphoreType.DMA,
        ],
    )


def _scale_body(degp_ref, x_ref, dinv_ref, u1_ref):
    deg = degp_ref[0, :, 0:1] + degp_ref[1, :, 0:1] + 1.0
    dinv = lax.rsqrt(deg)
    dinv_ref[...] = jnp.broadcast_to(dinv, (N, 16))
    u1_ref[...] = x_ref[...] * dinv


def _tc_scale(degp, x):
    return pl.pallas_call(
        _scale_body,
        out_shape=(
            jax.ShapeDtypeStruct((N, 16), jnp.float32),
            jax.ShapeDtypeStruct((N, F_IN), jnp.float32),
        ),
    )(degp, x)


def _mlp_body(s1p_ref, u1_ref, dinv_ref, w1_ref, b1_ref, w2_ref, u2_ref):
    dv = dinv_ref[:, 0:1]
    agg1 = dv * (s1p_ref[0] + s1p_ref[1] + u1_ref[...])
    h = jnp.maximum(
        jnp.dot(agg1, w1_ref[...], preferred_element_type=jnp.float32,
                ) + b1_ref[...],
        0.0)
    z = jnp.dot(h, w2_ref[...], preferred_element_type=jnp.float32)
    u2_ref[...] = dv * z


def _tc_mlp(s1p, u1, dinv, W1, b1_2d, W2p):
    return pl.pallas_call(
        _mlp_body,
        out_shape=jax.ShapeDtypeStruct((N, CP), jnp.float32),
    )(s1p, u1, dinv, W1, b1_2d, W2p)


def _loss_body(s2p_ref, u2_ref, dinv_ref, b2_ref, y_ref, maskf_ref, w_ref,
               out_ref):
    dv = dinv_ref[:, 0:1]
    logits = dv * (s2p_ref[0] + s2p_ref[1] + u2_ref[...]) + b2_ref[...]
    col = lax.broadcasted_iota(jnp.int32, (N, CP), 1)
    valid = col < C
    neg = jnp.where(valid, logits, -1e30)
    m = jnp.max(neg, axis=1, keepdims=True)
    ex = jnp.where(valid, jnp.exp(logits - m), 0.0)
    lse = jnp.log(jnp.sum(ex, axis=1, keepdims=True)) + m
    onehot = col == y_ref[...]
    picked = jnp.sum(jnp.where(onehot, logits, 0.0), axis=1, keepdims=True)
    wy = jnp.sum(jnp.where(onehot, w_ref[...], 0.0), axis=1, keepdims=True)
    nll = lse - picked
    wv = wy * maskf_ref[...]
    num = jnp.sum(nll * wv, axis=0, keepdims=True)
    den = jnp.sum(wv, axis=0, keepdims=True)
    out_ref[...] = num / den


def _tc_loss(s2p, u2, dinv, b2p, y2, maskf, wpad):
    return pl.pallas_call(
        _loss_body,
        out_shape=jax.ShapeDtypeStruct((1, 1), jnp.float32),
    )(s2p, u2, dinv, b2p, y2, maskf, wpad)


def kernel(x, edge_index, y, mask, weight, W1, b1, W2, b2):
    ei = edge_index.astype(jnp.int32).reshape(2, NW, NCHUNK, K)

    zeros16 = jnp.zeros((NP, 16), jnp.float32)
    zerosF = jnp.zeros((NP, F_IN), jnp.float32)
    zerosC = jnp.zeros((NP, CP), jnp.float32)
    onesK = jnp.ones((K, 16), jnp.float32)

    degp = _make_deg_kernel()(ei, onesK, zeros16)
    dinv, u1 = _tc_scale(degp, x)

    s1p = _make_agg_kernel(F_IN, 3)(u1, ei, zerosF)

    W2p = jnp.pad(W2, ((0, 0), (0, CP - C)))
    u2 = _tc_mlp(s1p, u1, dinv, W1, b1.reshape(1, HID), W2p)

    s2p = _make_agg_kernel(CP, 12)(u2, ei, zerosC)

    b2p = jnp.pad(b2, (0, CP - C)).reshape(1, CP)
    wpad = jnp.pad(weight, (0, CP - C)).reshape(1, CP)
    y2 = y.astype(jnp.int32).reshape(N, 1)
    maskf = mask.astype(jnp.float32).reshape(N, 1)

    loss = _tc_loss(s2p, u2, dinv, b2p, y2, maskf, wpad)
    return jnp.reshape(loss, ())
